# baseline (speedup 1.0000x reference)
import jax
import jax.numpy as jnp
from jax import lax
from jax.experimental import pallas as pl
from jax.experimental.pallas import tpu as pltpu
from jax.experimental.pallas import tpu_sc as plsc

_NC = 2
_NS = 16
_NW = _NC * _NS
_DB = 8
_TP = 4
_SC_NT = 128
_BT = 8


def _sc_body(nf, nch, x_ref, emb_ref, o_ref, emb_v, in_buf, out_buf,
             in_sem0, in_sem1, out_sem0, out_sem1):
    c = lax.axis_index("c")
    s = lax.axis_index("s")
    wid = s * _NC + c
    gd = (wid % (_NW // _TP)) * _DB
    tp = wid // (_NW // _TP)
    in_sems = (in_sem0, in_sem1)
    out_sems = (out_sem0, out_sem1)

    pltpu.sync_copy(emb_ref.at[pl.ds(gd, _DB)], emb_v)

    def in_copy(i, b):
        return pltpu.make_async_copy(
            x_ref.at[tp + i * _TP, pl.ds(gd, _DB)],
            in_buf.at[b], in_sems[b])

    def out_copy(i, b):
        return pltpu.make_async_copy(
            out_buf.at[b],
            o_ref.at[tp + i * _TP, pl.ds(gd, _DB)],
            out_sems[b])

    in_copy(0, 0).start()
    in_copy(1, 1).start()

    def step(i, b):
        in_copy(i, b).wait()

        @pl.when(i >= 2)
        def _():
            out_copy(i - 2, b).wait()

        def cbody(cc, carry):
            ds = pl.ds(cc * 16, 16)
            for r in range(_DB):
                out_buf[b, r, ds] = in_buf[b, r, ds] + emb_v[r, ds]
            return carry

        lax.fori_loop(0, nf // 16, cbody, 0)

        out_copy(i, b).start()

        @pl.when(i + 2 < nch)
        def _():
            in_copy(i + 2, b).start()

    def kbody(k, carry):
        step(k * 2, 0)
        step(k * 2 + 1, 1)
        return carry

    lax.fori_loop(0, nch // 2, kbody, 0)

    out_copy(nch - 2, 0).wait()
    out_copy(nch - 1, 1).wait()


def _tc_body(x_ref, emb_ref, alias_ref, o_ref):
    del alias_ref
    o_ref[...] = x_ref[...] + emb_ref[...]


def kernel(x, freqs, emb_weight):
    del freqs
    nt, nf, d = x.shape
    nch = _SC_NT // _TP
    assert d == _DB * (_NW // _TP) and nf % 16 == 0
    assert _SC_NT % (2 * _TP) == 0 and (nt - _SC_NT) % _BT == 0

    xt = jnp.transpose(x, (0, 2, 1))
    embt = jnp.transpose(emb_weight, (1, 0))

    body = lambda *refs: _sc_body(nf, nch, *refs)
    out_sc = pl.kernel(
        body,
        out_type=jax.ShapeDtypeStruct((nt, d, nf), x.dtype),
        mesh=plsc.VectorSubcoreMesh(core_axis_name="c", subcore_axis_name="s"),
        scratch_types=[
            pltpu.VMEM((_DB, nf), jnp.float32),
            pltpu.VMEM((2, _DB, nf), jnp.float32),
            pltpu.VMEM((2, _DB, nf), jnp.float32),
            pltpu.SemaphoreType.DMA,
            pltpu.SemaphoreType.DMA,
            pltpu.SemaphoreType.DMA,
            pltpu.SemaphoreType.DMA,
        ],
    )(xt, embt)

    off = _SC_NT // _BT
    outt = pl.pallas_call(
        _tc_body,
        grid=((nt - _SC_NT) // _BT,),
        in_specs=[
            pl.BlockSpec((_BT, d, nf), lambda i: (i + off, 0, 0)),
            pl.BlockSpec((d, nf), lambda i: (0, 0)),
            pl.BlockSpec(memory_space=pl.ANY),
        ],
        out_specs=pl.BlockSpec((_BT, d, nf), lambda i: (i + off, 0, 0)),
        out_shape=jax.ShapeDtypeStruct((nt, d, nf), x.dtype),
        input_output_aliases={2: 0},
    )(xt, embt, out_sc)
    return jnp.transpose(outt, (0, 2, 1))
